# Initial kernel scaffold; baseline (speedup 1.0000x reference)
import jax
import jax.numpy as jnp
from jax.experimental import pallas as pl


def kernel(x, edge_index, W1, b1, W2, b2):
    raise NotImplementedError("write your pallas kernel here")


# speedup vs baseline: 4.5551x; 4.5551x over previous
import functools

import jax
import jax.numpy as jnp
from jax import lax
from jax.experimental import pallas as pl
from jax.experimental.pallas import tpu as pltpu
from jax.experimental.pallas import tpu_sc as plsc

N_NODES = 10000
N_EDGES = 320000
D = 128
DH = 64

NC = 2
NS = 16
CH = 128
CHUNKS = 160
E_PAD = NS * CHUNKS * CH
N_ACC = NS * 640
STRIPE = 640
ROWS_B = 1000


def _lin1_body(x_ref, w_ref, b_ref, o1_ref, o2_ref):
    y = jnp.maximum(
        jnp.dot(x_ref[...], w_ref[...], preferred_element_type=jnp.float32)
        + b_ref[...], 0.0)
    o1_ref[...] = y[:, :DH]
    o2_ref[...] = y[:, DH:]


def _final_body(acc_ref, deg_ref, h1_ref, h2_ref, w_ref, b_ref, o_ref):
    agg = jnp.concatenate([acc_ref[0], acc_ref[1]], axis=1)
    deg = deg_ref[0, :, 0:1] + deg_ref[1, :, 0:1]
    deg = jnp.maximum(deg, 1.0)
    h = jnp.concatenate([h1_ref[...], h2_ref[...]], axis=1)
    z = agg / deg + h
    o_ref[...] = jnp.dot(z, w_ref[...], preferred_element_type=jnp.float32) + b_ref[...]


def _sc_body(src_hbm, dst_hbm, h1_hbm, h2_hbm, zacc_hbm, zdeg_hbm, ones_hbm,
             acc_out, deg_out,
             src_v, dst_v, rows_v, ones_v, acc_sh, deg_sh):
    c = lax.axis_index("c")
    s = lax.axis_index("s")
    pltpu.sync_copy(zacc_hbm, acc_sh.at[pl.ds(s * STRIPE, STRIPE)])
    pltpu.sync_copy(zdeg_hbm, deg_sh.at[pl.ds(s * STRIPE, STRIPE)])
    pltpu.sync_copy(src_hbm.at[s], src_v)
    pltpu.sync_copy(dst_hbm.at[s], dst_v)
    pltpu.sync_copy(ones_hbm, ones_v)
    plsc.subcore_barrier()

    def body(j, carry):
        @pl.when(c == 0)
        def _():
            pltpu.sync_copy(h1_hbm.at[src_v.at[j]], rows_v)

        @pl.when(c == 1)
        def _():
            pltpu.sync_copy(h2_hbm.at[src_v.at[j]], rows_v)

        pltpu.sync_copy(rows_v, acc_sh.at[dst_v.at[j]], add=True)

        @pl.when((j < CHUNKS // 2) == (c == 0))
        def _():
            pltpu.sync_copy(ones_v, deg_sh.at[dst_v.at[j]], add=True)

        return carry

    lax.fori_loop(0, CHUNKS, body, 0)
    plsc.subcore_barrier()
    pltpu.sync_copy(acc_sh.at[pl.ds(s * STRIPE, STRIPE)],
                    acc_out.at[c, pl.ds(s * STRIPE, STRIPE)])
    pltpu.sync_copy(deg_sh.at[pl.ds(s * STRIPE, STRIPE)],
                    deg_out.at[c, pl.ds(s * STRIPE, STRIPE)])


_sc_aggregate = functools.partial(
    pl.kernel,
    out_type=[
        jax.ShapeDtypeStruct((NC, N_ACC, DH), jnp.float32),
        jax.ShapeDtypeStruct((NC, N_ACC, 16), jnp.float32),
    ],
    mesh=plsc.VectorSubcoreMesh(core_axis_name="c", subcore_axis_name="s"),
    compiler_params=pltpu.CompilerParams(use_tc_tiling_on_sc=False),
    scratch_types=[
        pltpu.VMEM((CHUNKS, CH), jnp.int32),
        pltpu.VMEM((CHUNKS, CH), jnp.int32),
        pltpu.VMEM((CH, DH), jnp.float32),
        pltpu.VMEM((CH, 16), jnp.float32),
        pltpu.VMEM_SHARED((N_ACC, DH), jnp.float32),
        pltpu.VMEM_SHARED((N_ACC, 16), jnp.float32),
    ],
)(_sc_body)


def kernel(x, edge_index, W1, b1, W2, b2):
    src = edge_index[0].astype(jnp.int32)
    dst = edge_index[1].astype(jnp.int32)
    pad = E_PAD - N_EDGES
    src_p = jnp.concatenate([src, jnp.zeros((pad,), jnp.int32)]).reshape(NS, CHUNKS, CH)
    dst_p = jnp.concatenate([dst, jnp.full((pad,), N_NODES, jnp.int32)]).reshape(NS, CHUNKS, CH)

    h1, h2 = pl.pallas_call(
        _lin1_body,
        grid=(N_NODES // ROWS_B,),
        in_specs=[
            pl.BlockSpec((ROWS_B, D), lambda i: (i, 0)),
            pl.BlockSpec((D, D), lambda i: (0, 0)),
            pl.BlockSpec((1, D), lambda i: (0, 0)),
        ],
        out_specs=[
            pl.BlockSpec((ROWS_B, DH), lambda i: (i, 0)),
            pl.BlockSpec((ROWS_B, DH), lambda i: (i, 0)),
        ],
        out_shape=[
            jax.ShapeDtypeStruct((N_NODES, DH), jnp.float32),
            jax.ShapeDtypeStruct((N_NODES, DH), jnp.float32),
        ],
    )(x, W1.T, b1.reshape(1, D))

    zacc = jnp.zeros((STRIPE, DH), jnp.float32)
    zdeg = jnp.zeros((STRIPE, 16), jnp.float32)
    ones = jnp.ones((CH, 16), jnp.float32)
    acc2, deg2 = _sc_aggregate(src_p, dst_p, h1, h2, zacc, zdeg, ones)

    out = pl.pallas_call(
        _final_body,
        grid=(N_NODES // ROWS_B,),
        in_specs=[
            pl.BlockSpec((NC, ROWS_B, DH), lambda i: (0, i, 0)),
            pl.BlockSpec((NC, ROWS_B, 16), lambda i: (0, i, 0)),
            pl.BlockSpec((ROWS_B, DH), lambda i: (i, 0)),
            pl.BlockSpec((ROWS_B, DH), lambda i: (i, 0)),
            pl.BlockSpec((D, D), lambda i: (0, 0)),
            pl.BlockSpec((1, D), lambda i: (0, 0)),
        ],
        out_specs=pl.BlockSpec((ROWS_B, D), lambda i: (i, 0)),
        out_shape=jax.ShapeDtypeStruct((N_NODES, D), jnp.float32),
    )(acc2, deg2, h1, h2, W2.T, b2.reshape(1, D))
    return out


# speedup vs baseline: 5.9923x; 1.3155x over previous
import functools

import jax
import jax.numpy as jnp
from jax import lax
from jax.experimental import pallas as pl
from jax.experimental.pallas import tpu as pltpu
from jax.experimental.pallas import tpu_sc as plsc

N_NODES = 10000
N_EDGES = 320000
D = 128
DH = 64

NC = 2
NS = 16
CH = 128
CHUNKS = 160
E_PAD = NS * CHUNKS * CH
N_ACC = NS * 640
STRIPE = 640
ROWS_B = 1000


def _lin1_body(x_ref, w_ref, b_ref, o1_ref, o2_ref):
    y = jnp.maximum(
        jnp.dot(x_ref[...], w_ref[...], preferred_element_type=jnp.float32)
        + b_ref[...], 0.0)
    o1_ref[...] = y[:, :DH]
    o2_ref[...] = y[:, DH:]


def _final_body(acc_ref, deg_ref, h1_ref, h2_ref, w_ref, b_ref, o_ref):
    agg = jnp.concatenate([acc_ref[0], acc_ref[1]], axis=1)
    deg = deg_ref[0, :, 0:1] + deg_ref[1, :, 0:1]
    deg = jnp.maximum(deg, 1.0)
    h = jnp.concatenate([h1_ref[...], h2_ref[...]], axis=1)
    z = agg / deg + h
    o_ref[...] = jnp.dot(z, w_ref[...], preferred_element_type=jnp.float32) + b_ref[...]


NBUF = 4
PREF = 2


def _sc_body(src_hbm, dst_hbm, h1_hbm, h2_hbm, zacc_hbm, zdeg_hbm, ones_hbm,
             acc_out, deg_out,
             src_v, dst_v, rows_v, ones_v, acc_sh, deg_sh, *sems):
    gsem = sems[:NBUF]
    ssem = sems[NBUF:]
    c = lax.axis_index("c")
    s = lax.axis_index("s")
    pltpu.sync_copy(zacc_hbm, acc_sh.at[pl.ds(s * STRIPE, STRIPE)])
    pltpu.sync_copy(zdeg_hbm, deg_sh.at[pl.ds(s * STRIPE, STRIPE)])
    pltpu.sync_copy(src_hbm.at[s], src_v)
    pltpu.sync_copy(dst_hbm.at[s], dst_v)
    pltpu.sync_copy(ones_hbm, ones_v)
    plsc.subcore_barrier()

    def start_gather(j, b):
        @pl.when(c == 0)
        def _():
            pltpu.async_copy(h1_hbm.at[src_v.at[j]], rows_v.at[b], gsem[b])

        @pl.when(c == 1)
        def _():
            pltpu.async_copy(h2_hbm.at[src_v.at[j]], rows_v.at[b], gsem[b])

    def wait_gather(j, b):
        pltpu.make_async_copy(h1_hbm.at[src_v.at[j]], rows_v.at[b],
                              gsem[b]).wait()

    def wait_scatter(j, b):
        pltpu.make_async_copy(rows_v.at[b], acc_sh.at[dst_v.at[j]],
                              ssem[b]).wait()

    def group(g, carry):
        for b in range(NBUF):
            v = g * NBUF + b

            @pl.when(v < CHUNKS)
            def _():
                @pl.when(v >= NBUF)
                def _():
                    wait_scatter(v - NBUF, b)
                start_gather(v, b)

            u = v - PREF
            ub = (b - PREF) % NBUF

            @pl.when((u >= 0) & (u < CHUNKS))
            def _():
                wait_gather(u, ub)
                pltpu.async_copy(rows_v.at[ub], acc_sh.at[dst_v.at[u]],
                                 ssem[ub], add=True)
                @pl.when((u < CHUNKS // 2) == (c == 0))
                def _():
                    pltpu.sync_copy(ones_v, deg_sh.at[dst_v.at[u]], add=True)

        return carry

    n_groups = (CHUNKS + PREF + NBUF - 1) // NBUF
    lax.fori_loop(0, n_groups, group, 0)
    for i in range(NBUF):
        j = CHUNKS - NBUF + i
        wait_scatter(j, j % NBUF)
    plsc.subcore_barrier()
    pltpu.sync_copy(acc_sh.at[pl.ds(s * STRIPE, STRIPE)],
                    acc_out.at[c, pl.ds(s * STRIPE, STRIPE)])
    pltpu.sync_copy(deg_sh.at[pl.ds(s * STRIPE, STRIPE)],
                    deg_out.at[c, pl.ds(s * STRIPE, STRIPE)])


_sc_aggregate = functools.partial(
    pl.kernel,
    out_type=[
        jax.ShapeDtypeStruct((NC, N_ACC, DH), jnp.float32),
        jax.ShapeDtypeStruct((NC, N_ACC, 16), jnp.float32),
    ],
    mesh=plsc.VectorSubcoreMesh(core_axis_name="c", subcore_axis_name="s"),
    compiler_params=pltpu.CompilerParams(use_tc_tiling_on_sc=False),
    scratch_types=[
        pltpu.VMEM((CHUNKS, CH), jnp.int32),
        pltpu.VMEM((CHUNKS, CH), jnp.int32),
        pltpu.VMEM((NBUF, CH, DH), jnp.float32),
        pltpu.VMEM((CH, 16), jnp.float32),
        pltpu.VMEM_SHARED((N_ACC, DH), jnp.float32),
        pltpu.VMEM_SHARED((N_ACC, 16), jnp.float32),
    ] + [pltpu.SemaphoreType.DMA] * (2 * NBUF),
)(_sc_body)


def kernel(x, edge_index, W1, b1, W2, b2):
    src = edge_index[0].astype(jnp.int32)
    dst = edge_index[1].astype(jnp.int32)
    pad = E_PAD - N_EDGES
    src_p = jnp.concatenate([src, jnp.zeros((pad,), jnp.int32)]).reshape(NS, CHUNKS, CH)
    dst_p = jnp.concatenate([dst, jnp.full((pad,), N_NODES, jnp.int32)]).reshape(NS, CHUNKS, CH)

    h1, h2 = pl.pallas_call(
        _lin1_body,
        grid=(N_NODES // ROWS_B,),
        in_specs=[
            pl.BlockSpec((ROWS_B, D), lambda i: (i, 0)),
            pl.BlockSpec((D, D), lambda i: (0, 0)),
            pl.BlockSpec((1, D), lambda i: (0, 0)),
        ],
        out_specs=[
            pl.BlockSpec((ROWS_B, DH), lambda i: (i, 0)),
            pl.BlockSpec((ROWS_B, DH), lambda i: (i, 0)),
        ],
        out_shape=[
            jax.ShapeDtypeStruct((N_NODES, DH), jnp.float32),
            jax.ShapeDtypeStruct((N_NODES, DH), jnp.float32),
        ],
    )(x, W1.T, b1.reshape(1, D))

    zacc = jnp.zeros((STRIPE, DH), jnp.float32)
    zdeg = jnp.zeros((STRIPE, 16), jnp.float32)
    ones = jnp.ones((CH, 16), jnp.float32)
    acc2, deg2 = _sc_aggregate(src_p, dst_p, h1, h2, zacc, zdeg, ones)

    out = pl.pallas_call(
        _final_body,
        grid=(N_NODES // ROWS_B,),
        in_specs=[
            pl.BlockSpec((NC, ROWS_B, DH), lambda i: (0, i, 0)),
            pl.BlockSpec((NC, ROWS_B, 16), lambda i: (0, i, 0)),
            pl.BlockSpec((ROWS_B, DH), lambda i: (i, 0)),
            pl.BlockSpec((ROWS_B, DH), lambda i: (i, 0)),
            pl.BlockSpec((D, D), lambda i: (0, 0)),
            pl.BlockSpec((1, D), lambda i: (0, 0)),
        ],
        out_specs=pl.BlockSpec((ROWS_B, D), lambda i: (i, 0)),
        out_shape=jax.ShapeDtypeStruct((N_NODES, D), jnp.float32),
    )(acc2, deg2, h1, h2, W2.T, b2.reshape(1, D))
    return out


# speedup vs baseline: 6.0937x; 1.0169x over previous
import functools

import jax
import jax.numpy as jnp
from jax import lax
from jax.experimental import pallas as pl
from jax.experimental.pallas import tpu as pltpu
from jax.experimental.pallas import tpu_sc as plsc

N_NODES = 10000
N_EDGES = 320000
D = 128
DH = 64

NC = 2
NS = 16
CH = 128
CHUNKS = 160
E_PAD = NS * CHUNKS * CH
N_ACC = NS * 640
STRIPE = 640
ROWS_B = 1000


def _lin1_body(x_ref, w_ref, b_ref, o1_ref, o2_ref):
    y = jnp.maximum(
        jnp.dot(x_ref[...], w_ref[...], preferred_element_type=jnp.float32)
        + b_ref[...], 0.0)
    o1_ref[...] = y[:, :DH]
    o2_ref[...] = y[:, DH:]


def _final_body(acc_ref, deg_ref, h1_ref, h2_ref, w_ref, b_ref, o_ref):
    agg = jnp.concatenate([acc_ref[0], acc_ref[1]], axis=1)
    deg = deg_ref[0, :, 0:1] + deg_ref[1, :, 0:1]
    deg = jnp.maximum(deg, 1.0)
    h = jnp.concatenate([h1_ref[...], h2_ref[...]], axis=1)
    z = agg / deg + h
    o_ref[...] = jnp.dot(z, w_ref[...], preferred_element_type=jnp.float32) + b_ref[...]


NBUF = 4
PREF = 3


def _sc_body(src_hbm, dst_hbm, h1_hbm, h2_hbm, zacc_hbm, zdeg_hbm, ones_hbm,
             acc_out, deg_out,
             src_v, dst_v, rows_v, ones_v, acc_sh, deg_sh, *sems):
    gsem = sems[:NBUF]
    ssem = sems[NBUF:2 * NBUF]
    dsem = sems[2 * NBUF]
    c = lax.axis_index("c")
    s = lax.axis_index("s")
    pltpu.sync_copy(zacc_hbm, acc_sh.at[pl.ds(s * STRIPE, STRIPE)])
    pltpu.sync_copy(zdeg_hbm, deg_sh.at[pl.ds(s * STRIPE, STRIPE)])
    pltpu.sync_copy(src_hbm.at[s], src_v)
    pltpu.sync_copy(dst_hbm.at[s], dst_v)
    pltpu.sync_copy(ones_hbm, ones_v)
    plsc.subcore_barrier()

    def start_gather(j, b):
        @pl.when(c == 0)
        def _():
            pltpu.async_copy(h1_hbm.at[src_v.at[j]], rows_v.at[b], gsem[b])

        @pl.when(c == 1)
        def _():
            pltpu.async_copy(h2_hbm.at[src_v.at[j]], rows_v.at[b], gsem[b])

    def wait_gather(j, b):
        pltpu.make_async_copy(h1_hbm.at[src_v.at[j]], rows_v.at[b],
                              gsem[b]).wait()

    def wait_scatter(j, b):
        pltpu.make_async_copy(rows_v.at[b], acc_sh.at[dst_v.at[j]],
                              ssem[b]).wait()

    def group(g, carry):
        for b in range(NBUF):
            v = g * NBUF + b

            @pl.when(v < CHUNKS)
            def _():
                @pl.when(v >= NBUF)
                def _():
                    wait_scatter(v - NBUF, b)
                start_gather(v, b)

            u = v - PREF
            ub = (b - PREF) % NBUF

            @pl.when((u >= 0) & (u < CHUNKS))
            def _():
                wait_gather(u, ub)
                pltpu.async_copy(rows_v.at[ub], acc_sh.at[dst_v.at[u]],
                                 ssem[ub], add=True)
                @pl.when((u < CHUNKS // 2) == (c == 0))
                def _():
                    pltpu.async_copy(ones_v, deg_sh.at[dst_v.at[u]], dsem,
                                     add=True)

        return carry

    n_groups = (CHUNKS + PREF + NBUF - 1) // NBUF
    lax.fori_loop(0, n_groups, group, 0)
    for i in range(NBUF):
        j = CHUNKS - NBUF + i
        wait_scatter(j, j % NBUF)

    def drain_deg(j, carry):
        pltpu.make_async_copy(ones_v, deg_sh.at[dst_v.at[0]], dsem).wait()
        return carry

    lax.fori_loop(0, CHUNKS // 2, drain_deg, 0)
    plsc.subcore_barrier()
    pltpu.sync_copy(acc_sh.at[pl.ds(s * STRIPE, STRIPE)],
                    acc_out.at[c, pl.ds(s * STRIPE, STRIPE)])
    pltpu.sync_copy(deg_sh.at[pl.ds(s * STRIPE, STRIPE)],
                    deg_out.at[c, pl.ds(s * STRIPE, STRIPE)])


_sc_aggregate = functools.partial(
    pl.kernel,
    out_type=[
        jax.ShapeDtypeStruct((NC, N_ACC, DH), jnp.float32),
        jax.ShapeDtypeStruct((NC, N_ACC, 16), jnp.float32),
    ],
    mesh=plsc.VectorSubcoreMesh(core_axis_name="c", subcore_axis_name="s"),
    compiler_params=pltpu.CompilerParams(use_tc_tiling_on_sc=False),
    scratch_types=[
        pltpu.VMEM((CHUNKS, CH), jnp.int32),
        pltpu.VMEM((CHUNKS, CH), jnp.int32),
        pltpu.VMEM((NBUF, CH, DH), jnp.float32),
        pltpu.VMEM((CH, 16), jnp.float32),
        pltpu.VMEM_SHARED((N_ACC, DH), jnp.float32),
        pltpu.VMEM_SHARED((N_ACC, 16), jnp.float32),
    ] + [pltpu.SemaphoreType.DMA] * (2 * NBUF + 1),
)(_sc_body)


def kernel(x, edge_index, W1, b1, W2, b2):
    src = edge_index[0].astype(jnp.int32)
    dst = edge_index[1].astype(jnp.int32)
    pad = E_PAD - N_EDGES
    src_p = jnp.concatenate([src, jnp.zeros((pad,), jnp.int32)]).reshape(NS, CHUNKS, CH)
    dst_p = jnp.concatenate([dst, jnp.full((pad,), N_NODES, jnp.int32)]).reshape(NS, CHUNKS, CH)

    h1, h2 = pl.pallas_call(
        _lin1_body,
        grid=(N_NODES // ROWS_B,),
        in_specs=[
            pl.BlockSpec((ROWS_B, D), lambda i: (i, 0)),
            pl.BlockSpec((D, D), lambda i: (0, 0)),
            pl.BlockSpec((1, D), lambda i: (0, 0)),
        ],
        out_specs=[
            pl.BlockSpec((ROWS_B, DH), lambda i: (i, 0)),
            pl.BlockSpec((ROWS_B, DH), lambda i: (i, 0)),
        ],
        out_shape=[
            jax.ShapeDtypeStruct((N_NODES, DH), jnp.float32),
            jax.ShapeDtypeStruct((N_NODES, DH), jnp.float32),
        ],
    )(x, W1.T, b1.reshape(1, D))

    zacc = jnp.zeros((STRIPE, DH), jnp.float32)
    zdeg = jnp.zeros((STRIPE, 16), jnp.float32)
    ones = jnp.ones((CH, 16), jnp.float32)
    acc2, deg2 = _sc_aggregate(src_p, dst_p, h1, h2, zacc, zdeg, ones)

    out = pl.pallas_call(
        _final_body,
        grid=(N_NODES // ROWS_B,),
        in_specs=[
            pl.BlockSpec((NC, ROWS_B, DH), lambda i: (0, i, 0)),
            pl.BlockSpec((NC, ROWS_B, 16), lambda i: (0, i, 0)),
            pl.BlockSpec((ROWS_B, DH), lambda i: (i, 0)),
            pl.BlockSpec((ROWS_B, DH), lambda i: (i, 0)),
            pl.BlockSpec((D, D), lambda i: (0, 0)),
            pl.BlockSpec((1, D), lambda i: (0, 0)),
        ],
        out_specs=pl.BlockSpec((ROWS_B, D), lambda i: (i, 0)),
        out_shape=jax.ShapeDtypeStruct((N_NODES, D), jnp.float32),
    )(acc2, deg2, h1, h2, W2.T, b2.reshape(1, D))
    return out


# speedup vs baseline: 6.2499x; 1.0256x over previous
import functools

import jax
import jax.numpy as jnp
from jax import lax
from jax.experimental import pallas as pl
from jax.experimental.pallas import tpu as pltpu
from jax.experimental.pallas import tpu_sc as plsc

N_NODES = 10000
N_EDGES = 320000
D = 128
DH = 64

NC = 2
NS = 16
CH = 128
CHUNKS = 160
E_PAD = NS * CHUNKS * CH
N_ACC = NS * 640
STRIPE = 640
ROWS_B = 1000


def _lin1_body(x_ref, w_ref, b_ref, o1_ref, o2_ref):
    y = jnp.maximum(
        jnp.dot(x_ref[...], w_ref[...], preferred_element_type=jnp.float32)
        + b_ref[...], 0.0)
    o1_ref[...] = y[:, :DH]
    o2_ref[...] = y[:, DH:]


def _final_body(acc_ref, deg_ref, h1_ref, h2_ref, w_ref, b_ref, o_ref):
    agg = jnp.concatenate([acc_ref[0], acc_ref[1]], axis=1)
    deg = deg_ref[0, :, 0:1] + deg_ref[1, :, 0:1]
    deg = jnp.maximum(deg, 1.0)
    h = jnp.concatenate([h1_ref[...], h2_ref[...]], axis=1)
    z = agg / deg + h
    o_ref[...] = jnp.dot(z, w_ref[...], preferred_element_type=jnp.float32) + b_ref[...]


NBUF = 8
PREF = 4
SEG = 16
NSEG = CHUNKS // SEG


def _sc_body(src_hbm, dst_hbm, h1_hbm, h2_hbm, zacc_hbm, zdeg_hbm, ones_hbm,
             acc_out, deg_out,
             src_v, dst_v, rows_v, ones_v, acc_sh, deg_sh, *sems):
    gsem = sems[:NBUF]
    ssem = sems[NBUF:2 * NBUF]
    dsem = sems[2 * NBUF]
    isem = sems[2 * NBUF + 1]
    c = lax.axis_index("c")
    s = lax.axis_index("s")
    pltpu.sync_copy(zacc_hbm, acc_sh.at[pl.ds(s * STRIPE, STRIPE)])
    pltpu.sync_copy(zdeg_hbm, deg_sh.at[pl.ds(s * STRIPE, STRIPE)])
    pltpu.sync_copy(ones_hbm, ones_v)

    def stage_seg(k, kb):
        pltpu.async_copy(src_hbm.at[s, k], src_v.at[kb], isem)
        pltpu.async_copy(dst_hbm.at[s, k], dst_v.at[kb], isem)

    def wait_seg(k, kb):
        pltpu.make_async_copy(src_hbm.at[s, k], src_v.at[kb], isem).wait()
        pltpu.make_async_copy(dst_hbm.at[s, k], dst_v.at[kb], isem).wait()

    stage_seg(0, 0)
    plsc.subcore_barrier()

    def start_gather(j, b):
        sb = (j // SEG) % 2
        off = j % SEG

        @pl.when(c == 0)
        def _():
            pltpu.async_copy(h1_hbm.at[src_v.at[sb, off]], rows_v.at[b],
                             gsem[b])

        @pl.when(c == 1)
        def _():
            pltpu.async_copy(h2_hbm.at[src_v.at[sb, off]], rows_v.at[b],
                             gsem[b])

    def wait_gather(b):
        pltpu.make_async_copy(h1_hbm.at[src_v.at[0, 0]], rows_v.at[b],
                              gsem[b]).wait()

    def wait_scatter(b):
        pltpu.make_async_copy(rows_v.at[b], acc_sh.at[dst_v.at[0, 0]],
                              ssem[b]).wait()

    def group(g, carry):
        for b in range(NBUF):
            v = g * NBUF + b

            @pl.when(v < CHUNKS)
            def _():
                @pl.when(v % SEG == 0)
                def _():
                    wait_seg(v // SEG, (v // SEG) % 2)

                @pl.when(v >= NBUF)
                def _():
                    wait_scatter(b)
                start_gather(v, b)

                @pl.when((v % SEG == NBUF) & (v + SEG - NBUF < CHUNKS))
                def _():
                    stage_seg(v // SEG + 1, (v // SEG + 1) % 2)

            u = v - PREF
            ub = (b - PREF) % NBUF

            @pl.when((u >= 0) & (u < CHUNKS))
            def _():
                wait_gather(ub)
                usb = (u // SEG) % 2
                uoff = u % SEG
                pltpu.async_copy(rows_v.at[ub], acc_sh.at[dst_v.at[usb, uoff]],
                                 ssem[ub], add=True)
                @pl.when((u < CHUNKS // 2) == (c == 0))
                def _():
                    pltpu.async_copy(ones_v, deg_sh.at[dst_v.at[usb, uoff]],
                                     dsem, add=True)

        return carry

    n_groups = (CHUNKS + PREF + NBUF - 1) // NBUF
    lax.fori_loop(0, n_groups, group, 0)
    for b in range(NBUF):
        wait_scatter(b)
    pltpu.make_async_copy(deg_out.at[c], deg_sh, dsem).wait()
    plsc.subcore_barrier()
    pltpu.sync_copy(acc_sh.at[pl.ds(s * STRIPE, STRIPE)],
                    acc_out.at[c, pl.ds(s * STRIPE, STRIPE)])
    pltpu.sync_copy(deg_sh.at[pl.ds(s * STRIPE, STRIPE)],
                    deg_out.at[c, pl.ds(s * STRIPE, STRIPE)])


_sc_aggregate = functools.partial(
    pl.kernel,
    out_type=[
        jax.ShapeDtypeStruct((NC, N_ACC, DH), jnp.float32),
        jax.ShapeDtypeStruct((NC, N_ACC, 16), jnp.float32),
    ],
    mesh=plsc.VectorSubcoreMesh(core_axis_name="c", subcore_axis_name="s"),
    compiler_params=pltpu.CompilerParams(use_tc_tiling_on_sc=False),
    scratch_types=[
        pltpu.VMEM((2, SEG, CH), jnp.int32),
        pltpu.VMEM((2, SEG, CH), jnp.int32),
        pltpu.VMEM((NBUF, CH, DH), jnp.float32),
        pltpu.VMEM((CH, 16), jnp.float32),
        pltpu.VMEM_SHARED((N_ACC, DH), jnp.float32),
        pltpu.VMEM_SHARED((N_ACC, 16), jnp.float32),
    ] + [pltpu.SemaphoreType.DMA] * (2 * NBUF + 2),
)(_sc_body)


def kernel(x, edge_index, W1, b1, W2, b2):
    src = edge_index[0].astype(jnp.int32)
    dst = edge_index[1].astype(jnp.int32)
    pad = E_PAD - N_EDGES
    src_p = jnp.concatenate([src, jnp.zeros((pad,), jnp.int32)]).reshape(NS, NSEG, SEG, CH)
    dst_p = jnp.concatenate([dst, jnp.full((pad,), N_NODES, jnp.int32)]).reshape(NS, NSEG, SEG, CH)

    h1, h2 = pl.pallas_call(
        _lin1_body,
        grid=(N_NODES // ROWS_B,),
        in_specs=[
            pl.BlockSpec((ROWS_B, D), lambda i: (i, 0)),
            pl.BlockSpec((D, D), lambda i: (0, 0)),
            pl.BlockSpec((1, D), lambda i: (0, 0)),
        ],
        out_specs=[
            pl.BlockSpec((ROWS_B, DH), lambda i: (i, 0)),
            pl.BlockSpec((ROWS_B, DH), lambda i: (i, 0)),
        ],
        out_shape=[
            jax.ShapeDtypeStruct((N_NODES, DH), jnp.float32),
            jax.ShapeDtypeStruct((N_NODES, DH), jnp.float32),
        ],
    )(x, W1.T, b1.reshape(1, D))

    zacc = jnp.zeros((STRIPE, DH), jnp.float32)
    zdeg = jnp.zeros((STRIPE, 16), jnp.float32)
    ones = jnp.ones((CH, 16), jnp.float32)
    acc2, deg2 = _sc_aggregate(src_p, dst_p, h1, h2, zacc, zdeg, ones)

    out = pl.pallas_call(
        _final_body,
        grid=(N_NODES // ROWS_B,),
        in_specs=[
            pl.BlockSpec((NC, ROWS_B, DH), lambda i: (0, i, 0)),
            pl.BlockSpec((NC, ROWS_B, 16), lambda i: (0, i, 0)),
            pl.BlockSpec((ROWS_B, DH), lambda i: (i, 0)),
            pl.BlockSpec((ROWS_B, DH), lambda i: (i, 0)),
            pl.BlockSpec((D, D), lambda i: (0, 0)),
            pl.BlockSpec((1, D), lambda i: (0, 0)),
        ],
        out_specs=pl.BlockSpec((ROWS_B, D), lambda i: (i, 0)),
        out_shape=jax.ShapeDtypeStruct((N_NODES, D), jnp.float32),
    )(acc2, deg2, h1, h2, W2.T, b2.reshape(1, D))
    return out
